# baseline (speedup 1.0000x reference)
import functools

import jax
import jax.numpy as jnp
import numpy as np
from jax import lax
from jax.experimental import pallas as pl
from jax.experimental.pallas import tpu as pltpu, tpu_sc as plsc

N_NODES = 10000
N_EDGES = 320000
D = 128
REL_TYPES = 16
RSTR = REL_TYPES + 1

_info = plsc.get_sparse_core_info()
NC, NS, L = _info.num_cores, _info.num_subcores, _info.num_lanes
NW = NC * NS
PER_W = N_EDGES // NW
SUB = 80
CHUNK = 2 * SUB
N_FULL = PER_W // CHUNK
TAIL = PER_W - N_FULL * CHUNK
ASTR = L + 1


@functools.partial(
    pl.kernel,
    mesh=plsc.VectorSubcoreMesh(core_axis_name="c", subcore_axis_name="s"),
    out_type=jax.ShapeDtypeStruct((N_EDGES,), jnp.float32),
    scratch_types=[
        pltpu.VMEM((PER_W,), jnp.int32),
        pltpu.VMEM((PER_W,), jnp.int32),
        pltpu.VMEM((PER_W,), jnp.int32),
        pltpu.VMEM((CHUNK, D), jnp.bfloat16),
        pltpu.VMEM((CHUNK, D), jnp.bfloat16),
        pltpu.VMEM((CHUNK, D), jnp.bfloat16),
        pltpu.VMEM((CHUNK, D), jnp.bfloat16),
        pltpu.VMEM((D // 2 * RSTR,), jnp.float32),
        pltpu.VMEM((CHUNK * ASTR,), jnp.float32),
        pltpu.VMEM((PER_W,), jnp.float32),
        pltpu.VMEM_SHARED((N_NODES, D), jnp.bfloat16),
        pltpu.SemaphoreType.DMA,
        pltpu.SemaphoreType.DMA,
        pltpu.SemaphoreType.DMA,
        pltpu.SemaphoreType.DMA,
    ],
    compiler_params=pltpu.CompilerParams(needs_layout_passes=False,
                                         use_tc_tiling_on_sc=False),
)
def _distmult_sc(z_hbm, src_hbm, dst_hbm, rid_hbm, rel_hbm, out_hbm,
                 srci_v, dsti_v, rid_v, srcr0, srcr1, dstr0, dstr1,
                 rel_v, accs_v, out_v, z_sh, sem_s0, sem_s1, sem_d0, sem_d1):
    wid = lax.axis_index("s") * NC + lax.axis_index("c")
    base_w = wid * PER_W
    sid = lax.axis_index("s")
    zrows = N_NODES // NS
    pltpu.sync_copy(z_hbm.at[pl.ds(sid * zrows, zrows)],
                    z_sh.at[pl.ds(sid * zrows, zrows)])
    pltpu.sync_copy(rel_hbm, rel_v)
    pltpu.sync_copy(src_hbm.at[pl.ds(base_w, PER_W)], srci_v)
    pltpu.sync_copy(dst_hbm.at[pl.ds(base_w, PER_W)], dsti_v)
    pltpu.sync_copy(rid_hbm.at[pl.ds(base_w, PER_W)], rid_v)
    plsc.subcore_barrier()
    iota16 = lax.iota(jnp.int32, L)
    iota_astr = iota16 * ASTR
    pv = [(16 * m + iota16) * RSTR for m in range(D // 32)]

    srcr = (srcr0, srcr1)
    dstr = (dstr0, dstr1)
    sem_s = (sem_s0, sem_s1)
    sem_d = (sem_d0, sem_d1)

    def issue(base, b, nh):
        for h in range(nh):
            idx_s = srci_v.at[pl.ds(base + SUB * h, SUB)]
            idx_d = dsti_v.at[pl.ds(base + SUB * h, SUB)]
            dst_s = srcr[b].at[pl.ds(SUB * h, SUB)]
            dst_d = dstr[b].at[pl.ds(SUB * h, SUB)]
            pltpu.async_copy(z_sh.at[idx_s], dst_s, sem_s[b])
            pltpu.async_copy(z_sh.at[idx_d], dst_d, sem_d[b])

    def drain(b, nh):
        idx0 = srci_v.at[pl.ds(0, SUB)]
        for h in range(nh):
            pltpu.make_async_copy(
                z_sh.at[idx0], srcr[b].at[pl.ds(SUB * h, SUB)],
                sem_s[b]).wait()
            pltpu.make_async_copy(
                z_sh.at[idx0], dstr[b].at[pl.ds(SUB * h, SUB)],
                sem_d[b]).wait()

    def compute(base, b, n):
        @plsc.parallel_loop(0, n, unroll=8)
        def _edge_loop(k):
            kv = jnp.zeros((L,), jnp.int32) + (base + k)
            rid_b = plsc.load_gather(rid_v, [kv])
            acc0 = acc1 = None
            for m in range(D // 32):
                sv = srcr[b][k, pl.ds(m * 32, 32)]
                tv = dstr[b][k, pl.ds(m * 32, 32)]
                pe, po = plsc.unpack(sv * tv,
                                     format=plsc.PackFormat.INTERLEAVED)
                rp = plsc.load_gather(rel_v, [pv[m] + rid_b])
                re, ro = plsc.unpack(plsc.bitcast(rp, jnp.bfloat16),
                                     format=plsc.PackFormat.INTERLEAVED)
                if m == 0:
                    acc0 = pe * re
                    acc1 = po * ro
                else:
                    acc0 = acc0 + pe * re
                    acc1 = acc1 + po * ro
            accs_v[pl.ds(k * ASTR, L)] = acc0 + acc1

        @plsc.parallel_loop(0, n // L, unroll=1)
        def _reduce_loop(g):
            gb = g * (L * ASTR)
            parts = []
            for p4 in range(4):
                t = plsc.load_gather(accs_v, [gb + iota_astr + 4 * p4])
                for c in range(1, 4):
                    t = t + plsc.load_gather(
                        accs_v, [gb + iota_astr + 4 * p4 + c])
                parts.append(t)
            tot = (parts[0] + parts[1]) + (parts[2] + parts[3])
            sig = 1.0 / (1.0 + jnp.exp(-tot))
            out_v[pl.ds(base + g * L, L)] = sig

    issue(0, 0, 2)

    def body(kk, carry):
        c0 = kk * (2 * CHUNK)
        c1 = c0 + CHUNK
        issue(c1, 1, 2)
        drain(0, 2)
        compute(c0, 0, CHUNK)

        @pl.when(c0 + 2 * CHUNK < N_FULL * CHUNK)
        def _():
            issue(c0 + 2 * CHUNK, 0, 2)

        drain(1, 2)
        compute(c1, 1, CHUNK)
        return carry

    lax.fori_loop(0, N_FULL // 2, body, 0)
    issue(N_FULL * CHUNK, 0, 1)
    drain(0, 1)
    compute(N_FULL * CHUNK, 0, TAIL)
    pltpu.sync_copy(out_v, out_hbm.at[pl.ds(base_w, PER_W)])


def kernel(z, edge_index, rel_id, rel):
    src = edge_index[0].astype(jnp.int32)
    dst = edge_index[1].astype(jnp.int32)
    rid = rel_id.astype(jnp.int32)
    z_bf = z.astype(jnp.bfloat16)
    q = np.arange(D // 2)
    f_even = 32 * (q // 16) + 2 * (q % 16)
    rt = rel.astype(jnp.float32).T
    e16 = jax.lax.bitcast_convert_type(
        rt[f_even].astype(jnp.bfloat16), jnp.uint16).astype(jnp.uint32)
    o16 = jax.lax.bitcast_convert_type(
        rt[f_even + 1].astype(jnp.bfloat16), jnp.uint16).astype(jnp.uint32)
    packed = jax.lax.bitcast_convert_type(e16 | (o16 << 16), jnp.float32)
    rel_t = jnp.pad(packed, ((0, 0), (0, RSTR - REL_TYPES))).reshape(-1)
    return _distmult_sc(z_bf, src, dst, rid, rel_t)
